# Initial kernel scaffold; baseline (speedup 1.0000x reference)
import jax
import jax.numpy as jnp
from jax.experimental import pallas as pl


def kernel(x):
    raise NotImplementedError("write your pallas kernel here")


# speedup vs baseline: 3.0823x; 3.0823x over previous
import functools

import jax
import jax.numpy as jnp
from jax import lax
from jax.experimental import pallas as pl
from jax.experimental.pallas import tpu as pltpu
from jax.experimental.pallas import tpu_sc as plsc

ROWS = 128
COLS = 32768
KTOP = 64
LANES = 16
NBUCKET = 256
NCORES = 2
NSUB = 16
NWORK = NCORES * NSUB
ROWS_PER_W = ROWS // NWORK
NVREG = COLS // LANES
UNROLL = 4

_INT_MIN = -2147483648


def _key(xv):
    b = lax.bitcast_convert_type(xv, jnp.int32)
    m = lax.shift_right_logical(lax.shift_right_arithmetic(b, 31), 1)
    return lax.bitwise_xor(b, m)


def _topk_body(x_hbm, o_hbm, row_buf, hist, cand):
    cid = lax.axis_index("c")
    sid = lax.axis_index("s")
    wid = sid * NCORES + cid
    lane = lax.iota(jnp.int32, LANES)
    ones = jnp.ones((LANES,), jnp.int32)
    zeros16 = jnp.zeros((LANES,), jnp.int32)

    def _zero(i, carry):
        hist[pl.ds(i * LANES, LANES)] = zeros16
        return carry

    lax.fori_loop(0, NBUCKET, _zero, 0)

    for rr in range(ROWS_PER_W):
        row = wid * ROWS_PER_W + rr
        pltpu.sync_copy(x_hbm.at[row], row_buf)

        def _hist_body(i, carry):
            for uu in range(UNROLL):
                xv = row_buf[pl.ds((i * UNROLL + uu) * LANES, LANES)]
                s = _key(xv)
                d = lax.shift_right_arithmetic(s, 24)
                idx = (d + 128) * LANES + lane
                plsc.addupdate_scatter(hist, [idx], ones)
            return carry

        lax.fori_loop(0, NVREG // UNROLL, _hist_body, 0)

        def _scan_body(i, carry):
            s_acc, b, c_above, m_in_b = carry
            d = NBUCKET - 1 - i
            hvec = hist[pl.ds(d * LANES, LANES)]
            hist[pl.ds(d * LANES, LANES)] = zeros16
            cnt = jnp.sum(hvec)
            s_new = s_acc + cnt
            found = jnp.logical_and(s_acc < KTOP, s_new >= KTOP)
            b = jnp.where(found, d, b)
            c_above = jnp.where(found, s_acc, c_above)
            m_in_b = jnp.where(found, cnt, m_in_b)
            return (s_new, b, c_above, m_in_b)

        _, b, c_above, m_in_b = lax.fori_loop(
            0, NBUCKET, _scan_body,
            (jnp.int32(0), jnp.int32(0), jnp.int32(0), jnp.int32(0)))
        r = KTOP - c_above
        b_signed = b - 128

        def _compact_body(i, ptr_vec):
            for uu in range(UNROLL):
                xv = row_buf[pl.ds((i * UNROLL + uu) * LANES, LANES)]
                s = _key(xv)
                d = lax.shift_right_arithmetic(s, 24)
                msk = d == b_signed
                cs = plsc.cumsum(msk.astype(jnp.int32))
                idx = ptr_vec + cs - 1
                plsc.store_scatter(cand, [idx], s, mask=msk)
                ptr_vec = ptr_vec + plsc.all_reduce_population_count(msk)
            return ptr_vec

        ptr_vec = lax.fori_loop(0, NVREG // UNROLL, _compact_body,
                                jnp.zeros((LANES,), jnp.int32))
        plsc.store_scatter(cand, [ptr_vec + lane],
                           jnp.full((LANES,), _INT_MIN, jnp.int32))

        nv = lax.div(m_in_b + (LANES - 1), LANES)

        def _count_ge(t_scalar):
            t_splat = jnp.full((LANES,), t_scalar, jnp.int32)

            def _cbody(i, acc):
                cv = cand[pl.ds(i * LANES, LANES)]
                return acc + plsc.all_reduce_population_count(cv >= t_splat)

            acc = lax.fori_loop(0, nv, _cbody, zeros16)
            return jnp.max(acc)

        def _bit_body(j, prefix):
            trial = lax.bitwise_or(prefix, lax.shift_left(jnp.int32(1),
                                                          23 - j))
            cnt = _count_ge(trial)
            return jnp.where(cnt >= r, trial, prefix)

        t = lax.fori_loop(0, 24, _bit_body,
                          lax.shift_left(b_signed, 24))

        t_splat = jnp.full((LANES,), t, jnp.int32)

        def _cgt_body(i, acc):
            cv = cand[pl.ds(i * LANES, LANES)]
            return acc + plsc.all_reduce_population_count(cv > t_splat)

        c_gt = jnp.max(lax.fori_loop(0, nv, _cgt_body, zeros16))
        r_eq_splat = jnp.full((LANES,), r - c_gt, jnp.int32)

        def _out_body(i, cnt_vec):
            for uu in range(UNROLL):
                off = (i * UNROLL + uu) * LANES
                xv = row_buf[pl.ds(off, LANES)]
                s = _key(xv)
                eq = s == t_splat
                cs = plsc.cumsum(eq.astype(jnp.int32))
                keep_eq = jnp.logical_and(eq, (cnt_vec + cs) <= r_eq_splat)
                keep = jnp.logical_or(s > t_splat, keep_eq)
                val = jnp.maximum(xv, 0.0)
                row_buf[pl.ds(off, LANES)] = jnp.where(keep, val, 0.0)
                cnt_vec = cnt_vec + plsc.all_reduce_population_count(eq)
            return cnt_vec

        lax.fori_loop(0, NVREG // UNROLL, _out_body,
                      jnp.zeros((LANES,), jnp.int32))

        pltpu.sync_copy(row_buf, o_hbm.at[row])


@jax.jit
def kernel(x):
    mesh = plsc.VectorSubcoreMesh(core_axis_name="c", subcore_axis_name="s")
    f = pl.kernel(
        _topk_body,
        out_type=jax.ShapeDtypeStruct((ROWS, COLS), jnp.float32),
        mesh=mesh,
        compiler_params=pltpu.CompilerParams(needs_layout_passes=False),
        scratch_types=[
            pltpu.VMEM((COLS,), jnp.float32),
            pltpu.VMEM((NBUCKET * LANES,), jnp.int32),
            pltpu.VMEM((COLS + LANES,), jnp.int32),
        ],
    )
    return f(x)
